# baseline (speedup 1.0000x reference)
import jax
import jax.numpy as jnp
from jax.experimental import pallas as pl
from jax.experimental.pallas import tpu as pltpu

B = 2048
N0 = 10
N1 = 10
D = 256
R = 128
J = 8


def _fused_sage_kernel(h0_ref, h1f_ref, *rest):
    h2_refs = rest[:J]
    w0_ref, w1_ref, m_ref, out_ref = rest[J:]

    chunks = []
    for j in range(J):
        h2w = h2_refs[j][...]
        acc = h2w[:, 0:D]
        for k in range(1, N1):
            acc = acc + h2w[:, k * D:(k + 1) * D]
        chunks.append(acc)
    m2 = jnp.concatenate(chunks, axis=0) * (1.0 / N1)

    x1 = jnp.concatenate([h1f_ref[...], m2], axis=1)
    out1 = jnp.maximum(
        jnp.dot(x1, w0_ref[...], preferred_element_type=jnp.float32), 0.0)

    m1 = jnp.dot(m_ref[...], h1f_ref[...], preferred_element_type=jnp.float32)

    x0 = jnp.concatenate([h0_ref[...], m1], axis=1)
    out0 = jnp.maximum(
        jnp.dot(x0, w0_ref[...], preferred_element_type=jnp.float32), 0.0)

    mo1 = jnp.dot(m_ref[...], out1, preferred_element_type=jnp.float32)

    y = jnp.concatenate([out0, mo1], axis=1)
    out_ref[...] = jnp.dot(y, w1_ref[...], preferred_element_type=jnp.float32)


@jax.jit
def kernel(h0, h1, h2, W_self_0, W_neigh_0, W_self_1, W_neigh_1):
    h2w = h2.reshape(B * N0, N1 * D)
    w0 = jnp.concatenate([W_self_0, W_neigh_0], axis=0)
    w1 = jnp.concatenate([W_self_1, W_neigh_1], axis=0)
    m = jnp.repeat(jnp.eye(R, dtype=jnp.float32), N0, axis=1) * (1.0 / N0)

    rows = R * N0 // J
    grid = (B // R,)
    return pl.pallas_call(
        _fused_sage_kernel,
        grid=grid,
        in_specs=[
            pl.BlockSpec((R, D), lambda i: (i, 0)),
            pl.BlockSpec((R * N0, D), lambda i: (i, 0)),
        ] + [
            pl.BlockSpec((rows, N1 * D), lambda i, j=j: (i * J + j, 0))
            for j in range(J)
        ] + [
            pl.BlockSpec((2 * D, D), lambda i: (0, 0)),
            pl.BlockSpec((2 * D, D), lambda i: (0, 0)),
            pl.BlockSpec((R, R * N0), lambda i: (0, 0)),
        ],
        out_specs=pl.BlockSpec((R, D), lambda i: (i, 0)),
        out_shape=jax.ShapeDtypeStruct((B, D), jnp.float32),
        compiler_params=pltpu.CompilerParams(
            dimension_semantics=("arbitrary",)),
    )(h0, h1, *([h2w] * J), w0, w1, m)


# speedup vs baseline: 1.3894x; 1.3894x over previous
import functools

import jax
import jax.numpy as jnp
from jax import lax
from jax.experimental import pallas as pl
from jax.experimental.pallas import tpu as pltpu
from jax.experimental.pallas import tpu_sc as plsc

B = 2048
N0 = 10
N1 = 10
D = 256


NW = 32
OUT_PER_W = (B * N0) // NW
CH = 16
NCH = OUT_PER_W // CH
LANES = 16


def _sc_mean_body(h2_hbm, m2_hbm, buf0, buf1, out0, out1, rs0, rs1, ws0, ws1):
    wid = lax.axis_index("s") * 2 + lax.axis_index("c")
    in_base = wid * (OUT_PER_W * N1)
    out_base = wid * OUT_PER_W

    bufs = (buf0, buf1)
    outs = (out0, out1)
    rsems = (rs0, rs1)
    wsems = (ws0, ws1)

    pltpu.async_copy(h2_hbm.at[pl.ds(in_base, CH * N1)], buf0, rs0)
    pltpu.async_copy(h2_hbm.at[pl.ds(in_base + CH * N1, CH * N1)], buf1, rs1)

    def pair_body(gp, carry):
        for pb in range(2):
            buf, outb, rsem, wsem = bufs[pb], outs[pb], rsems[pb], wsems[pb]
            g = 2 * gp + pb
            pltpu.make_async_copy(
                h2_hbm.at[pl.ds(in_base, CH * N1)], buf, rsem).wait()
            @pl.when(g >= 2)
            def _():
                pltpu.make_async_copy(
                    outb, m2_hbm.at[pl.ds(out_base, CH)], wsem).wait()

            def row_body(r, c):
                base = r * N1
                for cb in range(D // LANES):
                    sl = pl.ds(cb * LANES, LANES)
                    acc = buf[base, sl]
                    for kk in range(1, N1):
                        acc = acc + buf[base + kk, sl]
                    outb[r, sl] = acc * (1.0 / N1)
                return c

            lax.fori_loop(0, CH, row_body, 0)

            pltpu.async_copy(
                outb, m2_hbm.at[pl.ds(out_base + g * CH, CH)], wsem)

            @pl.when(g + 2 < NCH)
            def _():
                pltpu.async_copy(
                    h2_hbm.at[pl.ds(in_base + (g + 2) * CH * N1, CH * N1)],
                    buf, rsem)
        return carry

    lax.fori_loop(0, NCH // 2, pair_body, 0)

    pltpu.make_async_copy(out0, m2_hbm.at[pl.ds(out_base, CH)], ws0).wait()
    pltpu.make_async_copy(out1, m2_hbm.at[pl.ds(out_base, CH)], ws1).wait()


def _sc_segment_mean(h2):
    mesh = plsc.VectorSubcoreMesh(core_axis_name="c", subcore_axis_name="s")
    f = functools.partial(
        pl.kernel,
        mesh=mesh,
        out_type=jax.ShapeDtypeStruct((B * N0, D), jnp.float32),
        scratch_types=[
            pltpu.VMEM((CH * N1, D), jnp.float32),
            pltpu.VMEM((CH * N1, D), jnp.float32),
            pltpu.VMEM((CH, D), jnp.float32),
            pltpu.VMEM((CH, D), jnp.float32),
            pltpu.SemaphoreType.DMA,
            pltpu.SemaphoreType.DMA,
            pltpu.SemaphoreType.DMA,
            pltpu.SemaphoreType.DMA,
        ],
    )(_sc_mean_body)
    return f(h2)


R = 128


def _tc_sage_kernel(h0_ref, h1f_ref, m2_ref, w0_ref, w1_ref, m_ref, out_ref):
    x1 = jnp.concatenate([h1f_ref[...], m2_ref[...]], axis=1)
    out1 = jnp.maximum(
        jnp.dot(x1, w0_ref[...], preferred_element_type=jnp.float32), 0.0)

    m1 = jnp.dot(m_ref[...], h1f_ref[...], preferred_element_type=jnp.float32)

    x0 = jnp.concatenate([h0_ref[...], m1], axis=1)
    out0 = jnp.maximum(
        jnp.dot(x0, w0_ref[...], preferred_element_type=jnp.float32), 0.0)

    mo1 = jnp.dot(m_ref[...], out1, preferred_element_type=jnp.float32)

    y = jnp.concatenate([out0, mo1], axis=1)
    out_ref[...] = jnp.dot(y, w1_ref[...], preferred_element_type=jnp.float32)


@jax.jit
def kernel(h0, h1, h2, W_self_0, W_neigh_0, W_self_1, W_neigh_1):
    m2 = _sc_segment_mean(h2)

    w0 = jnp.concatenate([W_self_0, W_neigh_0], axis=0)
    w1 = jnp.concatenate([W_self_1, W_neigh_1], axis=0)
    m = jnp.repeat(jnp.eye(R, dtype=jnp.float32), N0, axis=1) * (1.0 / N0)

    grid = (B // R,)
    return pl.pallas_call(
        _tc_sage_kernel,
        grid=grid,
        in_specs=[
            pl.BlockSpec((R, D), lambda i: (i, 0)),
            pl.BlockSpec((R * N0, D), lambda i: (i, 0)),
            pl.BlockSpec((R * N0, D), lambda i: (i, 0)),
            pl.BlockSpec((2 * D, D), lambda i: (0, 0)),
            pl.BlockSpec((2 * D, D), lambda i: (0, 0)),
            pl.BlockSpec((R, R * N0), lambda i: (0, 0)),
        ],
        out_specs=pl.BlockSpec((R, D), lambda i: (i, 0)),
        out_shape=jax.ShapeDtypeStruct((B, D), jnp.float32),
        compiler_params=pltpu.CompilerParams(
            dimension_semantics=("arbitrary",)),
    )(h0, h1, m2, w0, w1, m)
